# Initial kernel scaffold; baseline (speedup 1.0000x reference)
import jax
import jax.numpy as jnp
from jax.experimental import pallas as pl


def kernel(x_c, conv_A, conv_B, conv_D, conv_E, conv_H, alpha):
    raise NotImplementedError("write your pallas kernel here")


# speedup vs baseline: 1.0744x; 1.0744x over previous
import functools

import jax
import jax.numpy as jnp
from jax.experimental import pallas as pl
from jax.experimental.pallas import tpu as pltpu

B, N, F = 4, 2048, 64
NUM_HIDDEN = 5
TR = 512


def _soft(s, a):
    return jnp.where(s > a, s - a, jnp.where(s < -a, s + a, jnp.zeros_like(s)))


def _mm_body(m_ref, v_ref, o_ref):
    o_ref[0] = jnp.dot(m_ref[0], v_ref[0], preferred_element_type=jnp.float32)


def _mm_soft_body(m_ref, v_ref, a_ref, s_ref, z_ref):
    s = jnp.dot(m_ref[0], v_ref[0], preferred_element_type=jnp.float32)
    s_ref[0] = s
    z_ref[0] = _soft(s, a_ref[0])


def _mm_add_body(m_ref, v_ref, w_ref, o_ref):
    o_ref[0] = jnp.dot(m_ref[0], v_ref[0],
                       preferred_element_type=jnp.float32) + w_ref[0]


def _mm2_soft_body(m1_ref, v1_ref, m2_ref, v2_ref, a_ref, s_ref, z_ref):
    s = (jnp.dot(m1_ref[0], v1_ref[0], preferred_element_type=jnp.float32) +
         jnp.dot(m2_ref[0], v2_ref[0], preferred_element_type=jnp.float32))
    s_ref[0] = s
    z_ref[0] = _soft(s, a_ref[0])


_MAT = pl.BlockSpec((1, TR, N), lambda b, t: (b, t, 0))
_VEC = pl.BlockSpec((1, N, F), lambda b, t: (b, 0, 0))
_ROW = pl.BlockSpec((1, TR, F), lambda b, t: (b, t, 0))
_SCL = pl.BlockSpec(memory_space=pltpu.SMEM)
_GRID = (B, N // TR)
_OUT = jax.ShapeDtypeStruct((B, N, F), jnp.float32)


@jax.jit
def _mm(m, v):
    return pl.pallas_call(
        _mm_body, grid=_GRID,
        in_specs=[_MAT, _VEC], out_specs=_ROW, out_shape=_OUT,
    )(m, v)


@jax.jit
def _mm_soft(m, v, a):
    return pl.pallas_call(
        _mm_soft_body, grid=_GRID,
        in_specs=[_MAT, _VEC, _SCL], out_specs=(_ROW, _ROW),
        out_shape=(_OUT, _OUT),
    )(m, v, a)


@jax.jit
def _mm_add(m, v, w):
    return pl.pallas_call(
        _mm_add_body, grid=_GRID,
        in_specs=[_MAT, _VEC, _ROW], out_specs=_ROW, out_shape=_OUT,
    )(m, v, w)


@jax.jit
def _mm2_soft(m1, v1, m2, v2, a):
    return pl.pallas_call(
        _mm2_soft_body, grid=_GRID,
        in_specs=[_MAT, _VEC, _MAT, _VEC, _SCL], out_specs=(_ROW, _ROW),
        out_shape=(_OUT, _OUT),
    )(m1, v1, m2, v2, a)


@jax.jit
def kernel(x_c, conv_A, conv_B, conv_D, conv_E, conv_H, alpha):
    bx = _mm(conv_B, x_c)
    s, z = _mm_soft(conv_D, bx, alpha)
    for _ in range(NUM_HIDDEN - 1):
        y = _mm_add(conv_A, s, bx)
        s, z = _mm2_soft(conv_D, y, conv_E, z, alpha)
    return _mm(conv_H, s)
